# baseline (speedup 1.0000x reference)
import jax
import jax.numpy as jnp
from jax.experimental import pallas as pl
from jax.experimental.pallas import tpu as pltpu

D = 2048
F = 4096
E = 8
T = 2048
BT = 256
MAXB = T // BT + E - 1
PADN = MAXB * BT
BJ = 256
NJ = F // BJ

_CDIMS = (((0,), (0,)), ((), ()))


def _router_body(x_ref, rw_ref, rb_ref, tw_ref, ti_ref):
    l = jnp.dot(x_ref[...], rw_ref[...], preferred_element_type=jnp.float32)
    l = l + rb_ref[...]
    m = jnp.max(l, axis=1, keepdims=True)
    s = jnp.sum(jnp.exp(l - m), axis=1, keepdims=True)
    tw_ref[...] = 1.0 / s
    iota = jax.lax.broadcasted_iota(jnp.int32, l.shape, 1)
    ti_ref[...] = jnp.min(jnp.where(l >= m, iota, E), axis=1, keepdims=True)


def _router(flat, rw, rb):
    return pl.pallas_call(
        _router_body,
        out_shape=(
            jax.ShapeDtypeStruct((T, 1), jnp.float32),
            jax.ShapeDtypeStruct((T, 1), jnp.int32),
        ),
    )(flat, rw, rb.reshape(1, E))


def _moe_body(be_ref, xT_ref, wg_ref, wu_ref, wd_ref, tw_ref, o_ref):
    j = pl.program_id(0)
    b = pl.program_id(1)
    active = b < be_ref[MAXB]

    @pl.when(active)
    def _():
        cols = pl.ds(b * BT, BT)
        xT = xT_ref[:, cols]
        gT = jax.lax.dot_general(wg_ref[0], xT, _CDIMS,
                                 preferred_element_type=jnp.float32)
        uT = jax.lax.dot_general(wu_ref[0], xT, _CDIMS,
                                 preferred_element_type=jnp.float32)
        hT = (jax.nn.silu(gT) * uT).astype(jnp.bfloat16)
        pT = jax.lax.dot_general(wd_ref[0], hT, _CDIMS,
                                 preferred_element_type=jnp.float32)

        @pl.when(j == 0)
        def _():
            o_ref[:, cols] = pT

        @pl.when(j > 0)
        def _():
            o_ref[:, cols] = o_ref[:, cols] + pT

        @pl.when(j == NJ - 1)
        def _():
            o_ref[:, cols] = o_ref[:, cols] * tw_ref[:, cols]


def _grouped_mlp(x_pT, Wg, Wu, Wd, tw_p, be):
    grid_spec = pltpu.PrefetchScalarGridSpec(
        num_scalar_prefetch=1,
        grid=(NJ, MAXB),
        in_specs=[
            pl.BlockSpec((D, PADN), lambda j, b, be: (0, 0)),
            pl.BlockSpec((1, D, BJ), lambda j, b, be: (be[b], 0, j)),
            pl.BlockSpec((1, D, BJ), lambda j, b, be: (be[b], 0, j)),
            pl.BlockSpec((1, BJ, D), lambda j, b, be: (be[b], j, 0)),
            pl.BlockSpec((1, PADN), lambda j, b, be: (0, 0)),
        ],
        out_specs=pl.BlockSpec((D, PADN), lambda j, b, be: (0, 0)),
    )
    return pl.pallas_call(
        _moe_body,
        grid_spec=grid_spec,
        out_shape=jax.ShapeDtypeStruct((D, PADN), jnp.float32),
    )(be, x_pT, Wg, Wu, Wd, tw_p)


def kernel(hidden_states, router_W, router_b, Wg, Wu, Wd):
    B, S, _ = hidden_states.shape
    flat = hidden_states.reshape(T, D)

    tw, ti = _router(flat, router_W, router_b)
    topi = ti[:, 0]
    topw = tw[:, 0]

    order = jnp.argsort(topi, stable=True).astype(jnp.int32)
    counts = jnp.bincount(topi, length=E)
    nb = (counts + BT - 1) // BT
    cum_nb = jnp.cumsum(nb)
    nblocks = cum_nb[-1]
    pstart = (cum_nb - nb) * BT
    cstart = jnp.cumsum(counts) - counts
    e_sorted = topi[order]
    pos = (pstart[e_sorted] + jnp.arange(T) - cstart[e_sorted]).astype(jnp.int32)
    dest = jnp.zeros((T,), jnp.int32).at[order].set(pos)
    src = jnp.zeros((PADN,), jnp.int32).at[pos].set(order)
    bids = jnp.arange(MAXB)
    be_raw = jnp.sum(bids[:, None] >= cum_nb[None, :], axis=1)
    last_e = be_raw[nblocks - 1]
    be = jnp.where(bids < nblocks, be_raw, last_e)
    be = jnp.concatenate([be, nblocks[None]]).astype(jnp.int32)

    x_pT = flat.astype(jnp.bfloat16).T[:, src]
    tw_p = topw[src].reshape(1, PADN)

    y_pT = _grouped_mlp(x_pT, Wg.astype(jnp.bfloat16), Wu.astype(jnp.bfloat16),
                        Wd.astype(jnp.bfloat16), tw_p, be)
    out = y_pT[:, dest].T
    return out.reshape(B, S, D)


# speedup vs baseline: 1.2670x; 1.2670x over previous
import jax
import jax.numpy as jnp
from jax.experimental import pallas as pl
from jax.experimental.pallas import tpu as pltpu

D = 2048
F = 4096
E = 8
T = 2048
BT = 128
MAXB = T // BT + E - 1
PADN = MAXB * BT
BJ = 512
NJ = F // BJ


def _router_body(x_ref, rw_ref, rb_ref, tw_ref, ti_ref):
    l = jnp.dot(x_ref[...], rw_ref[...], preferred_element_type=jnp.float32)
    l = l + rb_ref[...]
    m = jnp.max(l, axis=1, keepdims=True)
    s = jnp.sum(jnp.exp(l - m), axis=1, keepdims=True)
    tw_ref[...] = 1.0 / s
    iota = jax.lax.broadcasted_iota(jnp.int32, l.shape, 1)
    ti_ref[...] = jnp.min(jnp.where(l >= m, iota, E), axis=1, keepdims=True)


def _router(flat, rw, rb):
    return pl.pallas_call(
        _router_body,
        out_shape=(
            jax.ShapeDtypeStruct((T, 1), jnp.float32),
            jax.ShapeDtypeStruct((T, 1), jnp.int32),
        ),
    )(flat, rw, rb.reshape(1, E))


def _moe_body(sp_ref, x_ref, wg_ref, wu_ref, wd_ref, tw_ref, o_ref):
    e = pl.program_id(0)
    j = pl.program_id(1)
    row0 = sp_ref[e]
    nblk = sp_ref[E + e]

    def blk(k, _):
        rows = pl.ds(pl.multiple_of(row0 + k * BT, BT), BT)
        x = x_ref[rows, :]
        g = jnp.dot(x, wg_ref[0], preferred_element_type=jnp.float32)
        u = jnp.dot(x, wu_ref[0], preferred_element_type=jnp.float32)
        h = (jax.nn.silu(g) * u).astype(jnp.bfloat16)
        part = jnp.dot(h, wd_ref[0], preferred_element_type=jnp.float32)

        @pl.when(j == 0)
        def _():
            o_ref[rows, :] = part

        @pl.when(j > 0)
        def _():
            o_ref[rows, :] = o_ref[rows, :] + part

        @pl.when(j == NJ - 1)
        def _():
            o_ref[rows, :] = o_ref[rows, :] * tw_ref[rows, :]

        return 0

    jax.lax.fori_loop(0, nblk, blk, 0)


def _grouped_mlp(x_p, Wg, Wu, Wd, tw_p, sp):
    grid_spec = pltpu.PrefetchScalarGridSpec(
        num_scalar_prefetch=1,
        grid=(E, NJ),
        in_specs=[
            pl.BlockSpec((PADN, D), lambda e, j, sp: (0, 0)),
            pl.BlockSpec((1, D, BJ), lambda e, j, sp: (e, 0, j)),
            pl.BlockSpec((1, D, BJ), lambda e, j, sp: (e, 0, j)),
            pl.BlockSpec((1, BJ, D), lambda e, j, sp: (e, j, 0)),
            pl.BlockSpec((PADN, 1), lambda e, j, sp: (0, 0)),
        ],
        out_specs=pl.BlockSpec((PADN, D), lambda e, j, sp: (0, 0)),
    )
    return pl.pallas_call(
        _moe_body,
        grid_spec=grid_spec,
        out_shape=jax.ShapeDtypeStruct((PADN, D), jnp.float32),
    )(sp, x_p, Wg, Wu, Wd, tw_p)


def kernel(hidden_states, router_W, router_b, Wg, Wu, Wd):
    B, S, _ = hidden_states.shape
    flat = hidden_states.reshape(T, D)

    tw, ti = _router(flat, router_W, router_b)
    topi = ti[:, 0]
    topw = tw[:, 0]

    order = jnp.argsort(topi, stable=True).astype(jnp.int32)
    counts = jnp.bincount(topi, length=E)
    nb = (counts + BT - 1) // BT
    cum_nb = jnp.cumsum(nb)
    pstart = (cum_nb - nb) * BT
    cstart = jnp.cumsum(counts) - counts
    e_sorted = topi[order]
    pos = (pstart[e_sorted] + jnp.arange(T) - cstart[e_sorted]).astype(jnp.int32)
    dest = jnp.zeros((T,), jnp.int32).at[order].set(pos)
    src = jnp.zeros((PADN,), jnp.int32).at[pos].set(order)
    sp = jnp.concatenate([pstart, nb]).astype(jnp.int32)

    x_p = flat.astype(jnp.bfloat16)[src]
    tw_p = topw[src].reshape(PADN, 1)

    y_p = _grouped_mlp(x_p, Wg.astype(jnp.bfloat16), Wu.astype(jnp.bfloat16),
                       Wd.astype(jnp.bfloat16), tw_p, sp)
    out = y_p[dest]
    return out.reshape(B, S, D)
